# baseline (speedup 1.0000x reference)
import functools

import jax
import jax.numpy as jnp
from jax import lax
from jax.experimental import pallas as pl
from jax.experimental.pallas import tpu as pltpu
from jax.experimental.pallas import tpu_sc as plsc

VOCAB = 32000
DIM = 4096
B = 8192
NC, NS = 2, 16
NW = NC * NS
BPW = B // NW
K = 8
NCH = BPW // K
NBUF = 3

_mesh = plsc.VectorSubcoreMesh(core_axis_name="c", subcore_axis_name="s")


@functools.partial(
    pl.kernel,
    mesh=_mesh,
    out_type=jax.ShapeDtypeStruct((4, 2048, DIM), jnp.float32),
    scratch_types=[
        pltpu.VMEM((BPW,), jnp.int32),
        pltpu.VMEM((K, DIM), jnp.float32),
        pltpu.VMEM((K, DIM), jnp.float32),
        pltpu.VMEM((K, DIM), jnp.float32),
        pltpu.SemaphoreType.DMA,
        pltpu.SemaphoreType.DMA,
        pltpu.SemaphoreType.DMA,
        pltpu.SemaphoreType.DMA,
        pltpu.SemaphoreType.DMA,
        pltpu.SemaphoreType.DMA,
    ],
)
def _emb(table_hbm, x_hbm, out_hbm, idx_v, buf0, buf1, buf2,
         gsem0, gsem1, gsem2, wsem0, wsem1, wsem2):
    bufs = (buf0, buf1, buf2)
    gsems = (gsem0, gsem1, gsem2)
    wsems = (wsem0, wsem1, wsem2)

    wid = lax.axis_index("s") * NC + lax.axis_index("c")
    q = wid // 8
    rofs = (wid % 8) * BPW
    pltpu.sync_copy(x_hbm.at[q, pl.ds(rofs, BPW)], idx_v)

    def g_copy(g, j):
        return pltpu.make_async_copy(
            table_hbm.at[idx_v.at[pl.ds(g * K, K)]], bufs[j], gsems[j])

    def w_copy(g, j):
        return pltpu.make_async_copy(
            bufs[j], out_hbm.at[q, pl.ds(rofs + g * K, K)], wsems[j])

    g_copy(0, 0).start()
    g_copy(1, 1).start()
    g_copy(0, 0).wait()
    w_copy(0, 0).start()
    g_copy(2, 2).start()

    def body(i, carry):
        for off, j in ((0, 1), (1, 2), (2, 0)):
            c = 3 * i + 1 + off
            g_copy(c, j).wait()
            w_copy(c, j).start()
            w_copy(c - 1, (j + 2) % 3).wait()
            g_copy(c + 2, (j + 2) % 3).start()
        return carry

    lax.fori_loop(0, (NCH - 5) // NBUF, body, 0)

    g_copy(NCH - 4, 1).wait()
    w_copy(NCH - 4, 1).start()
    w_copy(NCH - 5, 0).wait()
    g_copy(NCH - 2, 0).start()
    g_copy(NCH - 3, 2).wait()
    w_copy(NCH - 3, 2).start()
    w_copy(NCH - 4, 1).wait()
    g_copy(NCH - 1, 1).start()
    g_copy(NCH - 2, 0).wait()
    w_copy(NCH - 2, 0).start()
    w_copy(NCH - 3, 2).wait()
    g_copy(NCH - 1, 1).wait()
    w_copy(NCH - 1, 1).start()
    w_copy(NCH - 2, 0).wait()
    w_copy(NCH - 1, 1).wait()


def kernel(x, table):
    return _emb(table, x)


# speedup vs baseline: 1.0066x; 1.0066x over previous
import functools

import jax
import jax.numpy as jnp
from jax import lax
from jax.experimental import pallas as pl
from jax.experimental.pallas import tpu as pltpu
from jax.experimental.pallas import tpu_sc as plsc

VOCAB = 32000
DIM = 4096
B = 8192
NC, NS = 2, 16
NW = NC * NS
BPW = B // NW
K = 8
NCH = BPW // K
NBUF = 3

_mesh = plsc.VectorSubcoreMesh(core_axis_name="c", subcore_axis_name="s")


@functools.partial(
    pl.kernel,
    mesh=_mesh,
    out_type=jax.ShapeDtypeStruct((4, 2048, DIM), jnp.float32),
    scratch_types=[
        pltpu.VMEM((BPW,), jnp.int32),
        pltpu.VMEM((K, DIM), jnp.float32),
        pltpu.VMEM((K, DIM), jnp.float32),
        pltpu.VMEM((K, DIM), jnp.float32),
        pltpu.SemaphoreType.DMA,
        pltpu.SemaphoreType.DMA,
        pltpu.SemaphoreType.DMA,
        pltpu.SemaphoreType.DMA,
        pltpu.SemaphoreType.DMA,
        pltpu.SemaphoreType.DMA,
    ],
)
def _emb(table_hbm, x_hbm, out_hbm, idx_v, buf0, buf1, buf2,
         gsem0, gsem1, gsem2, wsem0, wsem1, wsem2):
    bufs = (buf0, buf1, buf2)
    gsems = (gsem0, gsem1, gsem2)
    wsems = (wsem0, wsem1, wsem2)

    wid = lax.axis_index("s") * NC + lax.axis_index("c")
    q = wid // 8
    rofs = (wid % 8) * BPW
    pltpu.sync_copy(x_hbm.at[q, pl.ds(rofs, BPW)], idx_v)

    def g_copy(g, j):
        return pltpu.make_async_copy(
            table_hbm.at[idx_v.at[pl.ds(g * K, K)]], bufs[j], gsems[j])

    def w_copy(g, j):
        return pltpu.make_async_copy(
            bufs[j], out_hbm.at[q, pl.ds(rofs + g * K, K)], wsems[j])

    g_copy(0, 0).start()
    g_copy(1, 1).start()
    g_copy(2, 2).start()

    def body(i, carry):
        for j in range(NBUF):
            c = NBUF * i + j
            g_copy(c, j).wait()
            w_copy(c, j).start()
            w_copy(c, j).wait()
            g_copy(jnp.minimum(c + NBUF, NCH - 1), j).start()
        return carry

    lax.fori_loop(0, (NCH - 2) // NBUF, body, 0)

    g_copy(NCH - 2, 0).wait()
    w_copy(NCH - 2, 0).start()
    g_copy(NCH - 1, 1).wait()
    w_copy(NCH - 1, 1).start()
    g_copy(NCH - 1, 2).wait()
    w_copy(NCH - 2, 0).wait()
    w_copy(NCH - 1, 1).wait()


def kernel(x, table):
    return _emb(table, x)
